# baseline (speedup 1.0000x reference)
import functools

import jax
import jax.numpy as jnp
from jax import lax
from jax.experimental import pallas as pl
from jax.experimental.pallas import tpu as pltpu
from jax.experimental.pallas import tpu_sc as plsc

FACTOR = 128
BATCH = 16384

NC, NS, L = 2, 16, 16
NW = NC * NS
B_PER_W = BATCH // NW
CHUNK = 128
N_CHUNKS = B_PER_W // CHUNK
FC = FACTOR // L
BLK = 8
N_BLK = CHUNK // BLK
N_ROWS = N_CHUNKS * N_BLK

_mesh = plsc.VectorSubcoreMesh(core_axis_name="c", subcore_axis_name="s")


@functools.partial(
    pl.kernel,
    out_type=jax.ShapeDtypeStruct((BATCH,), jnp.float32),
    mesh=_mesh,
    scratch_types=[
        pltpu.VMEM((N_CHUNKS, CHUNK), jnp.int32),
        pltpu.VMEM((N_CHUNKS, CHUNK), jnp.int32),
        pltpu.VMEM((2, CHUNK, FACTOR), jnp.float32),
        pltpu.VMEM((2, CHUNK, FACTOR), jnp.float32),
        pltpu.VMEM((N_ROWS, BLK * L), jnp.float32),
        pltpu.VMEM((N_ROWS, BLK * L), jnp.int32),
        pltpu.VMEM((B_PER_W,), jnp.float32),
        pltpu.VMEM((FACTOR,), jnp.float32),
        pltpu.VMEM((L,), jnp.float32),
        pltpu.VMEM_SHARED((NS * B_PER_W,), jnp.float32),
        pltpu.SemaphoreType.DMA,
        pltpu.SemaphoreType.DMA,
        pltpu.SemaphoreType.DMA,
        pltpu.SemaphoreType.DMA,
        pltpu.SemaphoreType.DMA,
        pltpu.SemaphoreType.DMA,
        pltpu.SemaphoreType.DMA,
        pltpu.SemaphoreType.DMA,
        pltpu.SemaphoreType.DMA,
        pltpu.SemaphoreType.DMA,
    ],
)
def _gmf_sc(user_hbm, item_hbm, utab_hbm, itab_hbm, w_hbm, b_hbm, out_hbm,
            idx_u, idx_i, u_rows, i_rows, acc, sidx, bfill, w_v, b_v, sh,
            sx0, sx1, sx2, sx3, sgu0, sgu1, sgi0, sgi1, sem_sc, sem_wb):
    sx = (sx0, sx1, sx2, sx3)
    sgu, sgi = (sgu0, sgu1), (sgi0, sgi1)
    cc = lax.axis_index("c")
    sid = lax.axis_index("s")
    wid = sid * NC + cc
    base = wid * B_PER_W
    shbase = sid * B_PER_W

    def issue_idx(c):
        cu = pltpu.async_copy(user_hbm.at[pl.ds(base + c * CHUNK, CHUNK)],
                              idx_u.at[c], sx[c])
        ci = pltpu.async_copy(item_hbm.at[pl.ds(base + c * CHUNK, CHUNK)],
                              idx_i.at[c], sx[c])
        return cu, ci

    def issue_gather(c):
        bb = c % 2
        gu = pltpu.async_copy(utab_hbm.at[idx_u.at[c]], u_rows.at[bb], sgu[bb])
        gi = pltpu.async_copy(itab_hbm.at[idx_i.at[c]], i_rows.at[bb], sgi[bb])
        return gu, gi

    idx_cps = {c: issue_idx(c) for c in range(N_CHUNKS)}
    cp_w = pltpu.async_copy(w_hbm, w_v, sem_wb)
    cp_b = pltpu.async_copy(b_hbm, b_v, sem_wb)
    idx_cps[0][0].wait()
    idx_cps[0][1].wait()
    gather_cps = {0: issue_gather(0)}
    scat_cps = []

    cp_w.wait()
    cp_b.wait()
    w_c = [w_v[pl.ds(L * j, L)] for j in range(FC)]
    b_vec = b_v[...]

    def fill(r, _):
        bfill[pl.ds(r * L, L)] = b_vec
        return 0
    lax.fori_loop(0, B_PER_W // L, fill, 0)
    pltpu.sync_copy(bfill, sh.at[pl.ds(shbase, B_PER_W)])

    def mkidx(rr, _):
        rowbase = shbase + rr * BLK
        for q in range(BLK):
            sidx[rr, pl.ds(q * L, L)] = jnp.full((L,), rowbase + q, jnp.int32)
        return 0
    lax.fori_loop(0, N_ROWS, mkidx, 0)

    for c in range(N_CHUNKS):
        bb = c % 2
        if c + 1 < N_CHUNKS:
            idx_cps[c + 1][0].wait()
            idx_cps[c + 1][1].wait()
            gather_cps[c + 1] = issue_gather(c + 1)
        gather_cps[c][0].wait()
        gather_cps[c][1].wait()

        ur, ir = u_rows.at[bb], i_rows.at[bb]

        def body(blk, _):
            e0 = blk * BLK
            for m in range(BLK):
                am = ur[e0 + m, pl.ds(0, L)] * ir[e0 + m, pl.ds(0, L)] * w_c[0]
                for j in range(1, FC):
                    am += (ur[e0 + m, pl.ds(L * j, L)]
                           * ir[e0 + m, pl.ds(L * j, L)] * w_c[j])
                acc[c * N_BLK + blk, pl.ds(m * L, L)] = am
            return 0

        lax.fori_loop(0, N_BLK, body, 0, unroll=4)

        for rr in range(c * N_BLK, (c + 1) * N_BLK):
            scat_cps.append(
                pltpu.async_copy(acc.at[rr], sh.at[sidx.at[rr]], sem_sc,
                                 add=True))

    for cp in scat_cps:
        cp.wait()
    pltpu.sync_copy(sh.at[pl.ds(shbase, B_PER_W)],
                    out_hbm.at[pl.ds(base, B_PER_W)])


def kernel(user, item, user_table, item_table, W, b):
    w_vec = W.reshape(FACTOR)
    b_vec = jnp.broadcast_to(b.reshape(()), (L,))
    return _gmf_sc(user, item, user_table, item_table, w_vec, b_vec)


# speedup vs baseline: 1.1535x; 1.1535x over previous
import functools

import jax
import jax.numpy as jnp
from jax import lax
from jax.experimental import pallas as pl
from jax.experimental.pallas import tpu as pltpu
from jax.experimental.pallas import tpu_sc as plsc

FACTOR = 128
BATCH = 16384

NC, NS, L = 2, 16, 16
NW = NC * NS
B_PER_W = BATCH // NW
CHUNK = 128
N_CHUNKS = B_PER_W // CHUNK
FC = FACTOR // L
BLK = 8
N_BLK = CHUNK // BLK
N_ROWS = N_CHUNKS * N_BLK

_mesh = plsc.VectorSubcoreMesh(core_axis_name="c", subcore_axis_name="s")


@functools.partial(
    pl.kernel,
    out_type=jax.ShapeDtypeStruct((BATCH,), jnp.float32),
    mesh=_mesh,
    scratch_types=[
        pltpu.VMEM((N_CHUNKS, CHUNK), jnp.int32),
        pltpu.VMEM((N_CHUNKS, CHUNK), jnp.int32),
        pltpu.VMEM((2, CHUNK, FACTOR), jnp.float32),
        pltpu.VMEM((2, CHUNK, FACTOR), jnp.float32),
        pltpu.VMEM((N_ROWS, BLK * L), jnp.float32),
        pltpu.VMEM((N_ROWS, BLK * L), jnp.int32),
        pltpu.VMEM((B_PER_W,), jnp.float32),
        pltpu.VMEM((FACTOR,), jnp.float32),
        pltpu.VMEM((L,), jnp.float32),
        pltpu.VMEM_SHARED((NS * B_PER_W,), jnp.float32),
        pltpu.SemaphoreType.DMA,
        pltpu.SemaphoreType.DMA,
        pltpu.SemaphoreType.DMA,
        pltpu.SemaphoreType.DMA,
        pltpu.SemaphoreType.DMA,
        pltpu.SemaphoreType.DMA,
        pltpu.SemaphoreType.DMA,
        pltpu.SemaphoreType.DMA,
        pltpu.SemaphoreType.DMA,
        pltpu.SemaphoreType.DMA,
    ],
)
def _gmf_sc(user_hbm, item_hbm, utab_hbm, itab_hbm, w_hbm, b_hbm, out_hbm,
            idx_u, idx_i, u_rows, i_rows, acc, sidx, bfill, w_v, b_v, sh,
            sx0, sx1, sx2, sx3, sgu0, sgu1, sgi0, sgi1, sem_sc, sem_wb):
    sx = (sx0, sx1, sx2, sx3)
    sgu, sgi = (sgu0, sgu1), (sgi0, sgi1)
    cc = lax.axis_index("c")
    sid = lax.axis_index("s")
    wid = sid * NC + cc
    base = wid * B_PER_W
    shbase = sid * B_PER_W

    def issue_idx(c):
        cu = pltpu.async_copy(user_hbm.at[pl.ds(base + c * CHUNK, CHUNK)],
                              idx_u.at[c], sx[c])
        ci = pltpu.async_copy(item_hbm.at[pl.ds(base + c * CHUNK, CHUNK)],
                              idx_i.at[c], sx[c])
        return cu, ci

    def issue_gather(c):
        bb = c % 2
        gu = pltpu.async_copy(utab_hbm.at[idx_u.at[c]], u_rows.at[bb], sgu[bb])
        gi = pltpu.async_copy(itab_hbm.at[idx_i.at[c]], i_rows.at[bb], sgi[bb])
        return gu, gi

    idx_cps = {c: issue_idx(c) for c in range(N_CHUNKS)}
    cp_w = pltpu.async_copy(w_hbm, w_v, sem_wb)
    cp_b = pltpu.async_copy(b_hbm, b_v, sem_wb)
    idx_cps[0][0].wait()
    idx_cps[0][1].wait()
    gather_cps = {0: issue_gather(0)}
    scat_cps = []

    cp_w.wait()
    cp_b.wait()
    w_c = [w_v[pl.ds(L * j, L)] for j in range(FC)]
    b_vec = b_v[...]

    def fill(r, _):
        bfill[pl.ds(r * L, L)] = b_vec
        return 0
    lax.fori_loop(0, B_PER_W // L, fill, 0)
    pltpu.sync_copy(bfill, sh.at[pl.ds(shbase, B_PER_W)])

    def mkidx(rr, _):
        rowbase = shbase + rr * BLK
        for q in range(BLK):
            sidx[rr, pl.ds(q * L, L)] = jnp.full((L,), rowbase + q, jnp.int32)
        return 0
    lax.fori_loop(0, N_ROWS, mkidx, 0)

    for c in range(N_CHUNKS):
        bb = c % 2
        if c + 1 < N_CHUNKS:
            idx_cps[c + 1][0].wait()
            idx_cps[c + 1][1].wait()
            gather_cps[c + 1] = issue_gather(c + 1)
        gather_cps[c][0].wait()
        gather_cps[c][1].wait()

        ur, ir = u_rows.at[bb], i_rows.at[bb]

        def body(blk, _):
            e0 = blk * BLK
            for m in range(BLK):
                am = ur[e0 + m, pl.ds(0, L)] * ir[e0 + m, pl.ds(0, L)] * w_c[0]
                for j in range(1, FC):
                    am += (ur[e0 + m, pl.ds(L * j, L)]
                           * ir[e0 + m, pl.ds(L * j, L)] * w_c[j])
                acc[c * N_BLK + blk, pl.ds(m * L, L)] = am
            return 0

        lax.fori_loop(0, N_BLK, body, 0)

        for rr in range(c * N_BLK, (c + 1) * N_BLK):
            scat_cps.append(
                pltpu.async_copy(acc.at[rr], sh.at[sidx.at[rr]], sem_sc,
                                 add=True))

    for cp in scat_cps:
        cp.wait()
    pltpu.sync_copy(sh.at[pl.ds(shbase, B_PER_W)],
                    out_hbm.at[pl.ds(base, B_PER_W)])


def kernel(user, item, user_table, item_table, W, b):
    w_vec = W.reshape(FACTOR)
    b_vec = jnp.broadcast_to(b.reshape(()), (L,))
    return _gmf_sc(user, item, user_table, item_table, w_vec, b_vec)


# speedup vs baseline: 1.1910x; 1.0325x over previous
import functools

import jax
import jax.numpy as jnp
from jax import lax
from jax.experimental import pallas as pl
from jax.experimental.pallas import tpu as pltpu
from jax.experimental.pallas import tpu_sc as plsc

FACTOR = 128
BATCH = 16384

NC, NS, L = 2, 16, 16
NW = NC * NS
B_PER_W = BATCH // NW
CHUNK = 128
N_CHUNKS = B_PER_W // CHUNK
FC = FACTOR // L
BLK = 8
N_BLK = CHUNK // BLK
N_ROWS = N_CHUNKS * N_BLK

_mesh = plsc.VectorSubcoreMesh(core_axis_name="c", subcore_axis_name="s")


@functools.partial(
    pl.kernel,
    out_type=jax.ShapeDtypeStruct((BATCH,), jnp.float32),
    mesh=_mesh,
    scratch_types=[
        pltpu.VMEM((N_CHUNKS, CHUNK), jnp.int32),
        pltpu.VMEM((N_CHUNKS, CHUNK), jnp.int32),
        pltpu.VMEM((2, CHUNK, FACTOR), jnp.float32),
        pltpu.VMEM((2, CHUNK, FACTOR), jnp.float32),
        pltpu.VMEM((N_ROWS, BLK * L), jnp.float32),
        pltpu.VMEM((N_ROWS, BLK * L), jnp.int32),
        pltpu.VMEM((B_PER_W,), jnp.float32),
        pltpu.VMEM((FACTOR,), jnp.float32),
        pltpu.VMEM((L,), jnp.float32),
        pltpu.VMEM_SHARED((NS * B_PER_W,), jnp.float32),
        pltpu.SemaphoreType.DMA,
        pltpu.SemaphoreType.DMA,
        pltpu.SemaphoreType.DMA,
        pltpu.SemaphoreType.DMA,
        pltpu.SemaphoreType.DMA,
        pltpu.SemaphoreType.DMA,
        pltpu.SemaphoreType.DMA,
        pltpu.SemaphoreType.DMA,
        pltpu.SemaphoreType.DMA,
        pltpu.SemaphoreType.DMA,
    ],
)
def _gmf_sc(user_hbm, item_hbm, utab_hbm, itab_hbm, w_hbm, b_hbm, out_hbm,
            idx_u, idx_i, u_rows, i_rows, acc, sidx, bfill, w_v, b_v, sh,
            sx0, sx1, sx2, sx3, sgu0, sgu1, sgi0, sgi1, sem_sc, sem_wb):
    sx = (sx0, sx1, sx2, sx3)
    sgu, sgi = (sgu0, sgu1), (sgi0, sgi1)
    cc = lax.axis_index("c")
    sid = lax.axis_index("s")
    wid = sid * NC + cc
    base = wid * B_PER_W
    shbase = sid * B_PER_W

    def issue_idx(c):
        cu = pltpu.async_copy(user_hbm.at[pl.ds(base + c * CHUNK, CHUNK)],
                              idx_u.at[c], sx[c])
        ci = pltpu.async_copy(item_hbm.at[pl.ds(base + c * CHUNK, CHUNK)],
                              idx_i.at[c], sx[c])
        return cu, ci

    def issue_gather(c):
        bb = c % 2
        gu = pltpu.async_copy(utab_hbm.at[idx_u.at[c]], u_rows.at[bb], sgu[bb])
        gi = pltpu.async_copy(itab_hbm.at[idx_i.at[c]], i_rows.at[bb], sgi[bb])
        return gu, gi

    idx_cps = {c: issue_idx(c) for c in range(N_CHUNKS)}
    cp_w = pltpu.async_copy(w_hbm, w_v, sem_wb)
    cp_b = pltpu.async_copy(b_hbm, b_v, sem_wb)
    idx_cps[0][0].wait()
    idx_cps[0][1].wait()
    gather_cps = {0: issue_gather(0)}
    for c in range(1, N_CHUNKS):
        idx_cps[c][0].wait()
        idx_cps[c][1].wait()

    cp_w.wait()
    cp_b.wait()
    w_c = [w_v[pl.ds(L * j, L)] for j in range(FC)]
    b_vec = b_v[...]

    def fill(r, _):
        bfill[pl.ds(r * L, L)] = b_vec
        return 0
    lax.fori_loop(0, B_PER_W // L, fill, 0)
    pltpu.sync_copy(bfill, sh.at[pl.ds(shbase, B_PER_W)])

    def mkidx(rr, _):
        rowbase = shbase + rr * BLK
        for q in range(BLK):
            sidx[rr, pl.ds(q * L, L)] = jnp.full((L,), rowbase + q, jnp.int32)
        return 0
    lax.fori_loop(0, N_ROWS, mkidx, 0)

    for c in range(N_CHUNKS):
        bb = c % 2
        if c + 1 < N_CHUNKS:
            gather_cps[c + 1] = issue_gather(c + 1)
        gather_cps[c][0].wait()
        gather_cps[c][1].wait()

        ur, ir = u_rows.at[bb], i_rows.at[bb]

        def body(blk, _):
            e0 = blk * BLK
            for m in range(BLK):
                am = ur[e0 + m, pl.ds(0, L)] * ir[e0 + m, pl.ds(0, L)] * w_c[0]
                for j in range(1, FC):
                    am += (ur[e0 + m, pl.ds(L * j, L)]
                           * ir[e0 + m, pl.ds(L * j, L)] * w_c[j])
                acc[c * N_BLK + blk, pl.ds(m * L, L)] = am
            rr = c * N_BLK + blk
            pltpu.async_copy(acc.at[rr], sh.at[sidx.at[rr]], sem_sc, add=True)
            return 0

        lax.fori_loop(0, N_BLK, body, 0)

    def drain(rr, _):
        pltpu.make_async_copy(acc.at[rr], sh.at[sidx.at[rr]], sem_sc).wait()
        return 0
    lax.fori_loop(0, N_ROWS, drain, 0)
    pltpu.sync_copy(sh.at[pl.ds(shbase, B_PER_W)],
                    out_hbm.at[pl.ds(base, B_PER_W)])


def kernel(user, item, user_table, item_table, W, b):
    w_vec = W.reshape(FACTOR)
    b_vec = jnp.broadcast_to(b.reshape(()), (L,))
    return _gmf_sc(user, item, user_table, item_table, w_vec, b_vec)
